# baseline (speedup 1.0000x reference)
import functools

import jax
import jax.numpy as jnp
from jax import lax
from jax.experimental import pallas as pl
from jax.experimental.pallas import tpu as pltpu
from jax.experimental.pallas import tpu_sc as plsc

N = 320000
IN_DIM = 128
HID = 128
HEAD = 4
NUM_SEG = 10000

BA = 2000
BC = 2000


def _att_body(res_ref, int_ref, m_ref, et_ref, out_ref):
    rb = res_ref[...].astype(jnp.bfloat16)
    P = jnp.dot(rb, m_ref[...], preferred_element_type=jnp.float32)
    ib = int_ref[...].astype(jnp.bfloat16)
    i4 = jnp.concatenate([ib, ib, ib, ib], axis=1)
    PI = (P.astype(jnp.bfloat16) * i4)
    out_ref[...] = jnp.dot(PI, et_ref[...],
                           preferred_element_type=jnp.float32)


def _ln(x, g, b, eps=1e-5):
    mu = jnp.mean(x, axis=-1, keepdims=True)
    xc = x - mu
    var = jnp.mean(xc * xc, axis=-1, keepdims=True)
    return xc * jax.lax.rsqrt(var + eps) * g + b


def _out_body(res_ref, al_ref, v_ref, w1_ref, b1_ref, w2_ref, b2_ref,
              g_ref, bt_ref, out_ref):
    res = res_ref[...]
    rb = res.astype(jnp.bfloat16)
    PV = jnp.dot(rb, v_ref[...], preferred_element_type=jnp.float32)
    al = al_ref[...]
    mo = (al[:, 0:1] * PV[:, 0:128] + al[:, 1:2] * PV[:, 128:256]
          + al[:, 2:3] * PV[:, 256:384] + al[:, 3:4] * PV[:, 384:512])
    g = g_ref[...]
    bt = bt_ref[...]
    x = _ln(mo + res, g, bt)
    h1 = jnp.dot(x.astype(jnp.bfloat16), w1_ref[...],
                 preferred_element_type=jnp.float32) + b1_ref[...]
    h1 = jnp.maximum(h1, 0.0)
    h2 = jnp.dot(h1.astype(jnp.bfloat16), w2_ref[...],
                 preferred_element_type=jnp.float32) + b2_ref[...]
    out_ref[...] = _ln(h2 + x, g, bt)


def _attention_logits(residue_h, inter_h, Mstack_bf, ET_bf):
    grid = (N // BA,)
    return pl.pallas_call(
        _att_body,
        grid=grid,
        in_specs=[
            pl.BlockSpec((BA, IN_DIM), lambda i: (i, 0)),
            pl.BlockSpec((BA, IN_DIM), lambda i: (i, 0)),
            pl.BlockSpec((IN_DIM, 4 * HID), lambda i: (0, 0)),
            pl.BlockSpec((4 * HID, HEAD), lambda i: (0, 0)),
        ],
        out_specs=pl.BlockSpec((BA, HEAD), lambda i: (i, 0)),
        out_shape=jax.ShapeDtypeStruct((N, HEAD), jnp.float32),
        compiler_params=pltpu.CompilerParams(
            dimension_semantics=("parallel",)),
    )(residue_h, inter_h, Mstack_bf, ET_bf)


def _output_block(residue_h, alpha, Vstack_bf, W1t_bf, b1r, W2t_bf, b2r,
                  gr, br):
    grid = (N // BC,)
    return pl.pallas_call(
        _out_body,
        grid=grid,
        in_specs=[
            pl.BlockSpec((BC, IN_DIM), lambda i: (i, 0)),
            pl.BlockSpec((BC, HEAD), lambda i: (i, 0)),
            pl.BlockSpec((IN_DIM, 4 * HID), lambda i: (0, 0)),
            pl.BlockSpec((HID, 2 * HID), lambda i: (0, 0)),
            pl.BlockSpec((1, 2 * HID), lambda i: (0, 0)),
            pl.BlockSpec((2 * HID, HID), lambda i: (0, 0)),
            pl.BlockSpec((1, HID), lambda i: (0, 0)),
            pl.BlockSpec((1, IN_DIM), lambda i: (0, 0)),
            pl.BlockSpec((1, IN_DIM), lambda i: (0, 0)),
        ],
        out_specs=pl.BlockSpec((BC, IN_DIM), lambda i: (i, 0)),
        out_shape=jax.ShapeDtypeStruct((N, IN_DIM), jnp.float32),
        compiler_params=pltpu.CompilerParams(
            dimension_semantics=("parallel",)),
    )(residue_h, alpha, Vstack_bf, W1t_bf, b1r, W2t_bf, b2r, gr, br)


NC = 2
NS = 16
LANES = 16
CH = 2000
SEGP = 40960
TOK_SC = N // NS
TOK_W = N // (NC * NS)
RED = SEGP // NS


def _sc_softmax_body(att_hbm, batch_hbm, alpha_hbm, denom, attc, bc, outc,
                     tmp, acc, shared):
    cid = lax.axis_index("c")
    sid = lax.axis_index("s")
    iot = lax.iota(jnp.int32, LANES)
    zero16 = jnp.zeros((LANES,), jnp.float32)

    def zero_body(i, _):
        denom[pl.ds(i * LANES, LANES)] = zero16
        return 0

    lax.fori_loop(0, SEGP // LANES, zero_body, 0)

    def chunk1(k, _):
        tok0 = sid * TOK_SC + k * CH
        pltpu.sync_copy(att_hbm.at[pl.ds(tok0, CH)], attc)
        pltpu.sync_copy(batch_hbm.at[pl.ds(tok0, CH)], bc.at[pl.ds(0, CH)])

        def vr(j, _):
            base = j * LANES
            b = bc[pl.ds(base, LANES)]
            bn = bc[pl.ds(base + 1, LANES)]
            is_end = (b != bn) | (iot == LANES - 1)
            is_mid_end = is_end & (iot != LANES - 1)
            b4 = b * HEAD
            bn4 = bn * HEAD
            ti = base + iot
            for h in range(HEAD):
                hv = jnp.full((LANES,), h, jnp.int32)
                e = jnp.exp(plsc.load_gather(attc, [ti, hv]))
                c = plsc.cumsum(e)
                plsc.addupdate_scatter(denom, [b4 + h], c, mask=is_end)
                plsc.addupdate_scatter(denom, [bn4 + h], -c, mask=is_mid_end)
            return 0

        lax.fori_loop(0, CH // LANES, vr, 0)
        return 0

    lax.fori_loop(0, TOK_SC // CH, chunk1, 0)

    pltpu.sync_copy(denom, shared.at[sid])
    plsc.subcore_barrier()
    s0 = sid * RED
    pltpu.sync_copy(shared.at[0, pl.ds(s0, RED)], acc)

    def red(u, _):
        pltpu.sync_copy(shared.at[u, pl.ds(s0, RED)], tmp)

        def addv(i, _):
            sl = pl.ds(i * LANES, LANES)
            acc[sl] += tmp[sl]
            return 0

        lax.fori_loop(0, RED // LANES, addv, 0)
        return 0

    lax.fori_loop(1, NS, red, 0)
    pltpu.sync_copy(acc, shared.at[0, pl.ds(s0, RED)])
    plsc.subcore_barrier()
    pltpu.sync_copy(shared.at[0], denom)

    wid = cid * NS + sid

    def chunk2(k, _):
        tok0 = wid * TOK_W + k * CH
        pltpu.sync_copy(att_hbm.at[pl.ds(tok0, CH)], attc)
        pltpu.sync_copy(batch_hbm.at[pl.ds(tok0, CH)], bc.at[pl.ds(0, CH)])

        def vr(j, _):
            base = j * LANES
            b = bc[pl.ds(base, LANES)]
            b4 = b * HEAD
            ti = base + iot
            for h in range(HEAD):
                hv = jnp.full((LANES,), h, jnp.int32)
                e = jnp.exp(plsc.load_gather(attc, [ti, hv]))
                d = plsc.load_gather(denom, [b4 + h])
                plsc.store_scatter(outc, [ti, hv], e / d)
            return 0

        lax.fori_loop(0, CH // LANES, vr, 0)
        pltpu.sync_copy(outc, alpha_hbm.at[pl.ds(tok0, CH)])
        return 0

    lax.fori_loop(0, TOK_W // CH, chunk2, 0)


def _sc_softmax(attT, batch):
    mesh = plsc.VectorSubcoreMesh(core_axis_name="c", subcore_axis_name="s")
    return pl.kernel(
        _sc_softmax_body,
        out_type=jax.ShapeDtypeStruct((N, HEAD), jnp.float32),
        mesh=mesh,
        scratch_types=[
            pltpu.VMEM((SEGP,), jnp.float32),
            pltpu.VMEM((CH, HEAD), jnp.float32),
            pltpu.VMEM((CH + LANES,), jnp.int32),
            pltpu.VMEM((CH, HEAD), jnp.float32),
            pltpu.VMEM((RED,), jnp.float32),
            pltpu.VMEM((RED,), jnp.float32),
            pltpu.VMEM_SHARED((NS, SEGP), jnp.float32),
        ],
        compiler_params=pltpu.CompilerParams(needs_layout_passes=False, use_tc_tiling_on_sc=False),
    )(attT, batch)


def kernel(residue_h, inter_h, Wq, Wk, Wv, Wc, W1, b1, W2, b2, gamma, beta,
           batch):
    scale = jnp.sqrt(jnp.float32(1280.0))
    Mstack = jnp.concatenate(
        [Wq[i].T @ Wk[i] for i in range(HEAD)], axis=1) / scale
    Vstack = jnp.concatenate(
        [Wv[i].T @ Wc[:, i * HID:(i + 1) * HID].T for i in range(HEAD)],
        axis=1)
    ET = jnp.repeat(jnp.eye(HEAD, dtype=jnp.float32), HID, axis=0)

    attT = _attention_logits(residue_h, inter_h,
                             Mstack.astype(jnp.bfloat16),
                             ET.astype(jnp.bfloat16))

    alpha = _sc_softmax(attT, batch)

    return _output_block(residue_h, alpha,
                         Vstack.astype(jnp.bfloat16),
                         W1.T.astype(jnp.bfloat16),
                         b1.reshape(1, -1),
                         W2.T.astype(jnp.bfloat16),
                         b2.reshape(1, -1),
                         gamma.reshape(1, -1),
                         beta.reshape(1, -1))


# speedup vs baseline: 1.4394x; 1.4394x over previous
import functools

import jax
import jax.numpy as jnp
from jax import lax
from jax.experimental import pallas as pl
from jax.experimental.pallas import tpu as pltpu
from jax.experimental.pallas import tpu_sc as plsc

N = 320000
IN_DIM = 128
HID = 128
HEAD = 4
NUM_SEG = 10000

BA = 3200
BC = 3200


def _att_body(res_ref, int_ref, m_ref, et_ref, out_ref):
    rb = res_ref[...].astype(jnp.bfloat16)
    P = jnp.dot(rb, m_ref[...], preferred_element_type=jnp.float32)
    ib = int_ref[...].astype(jnp.bfloat16)
    i4 = jnp.concatenate([ib, ib, ib, ib], axis=1)
    PI = (P.astype(jnp.bfloat16) * i4)
    out_ref[...] = jax.lax.dot_general(
        et_ref[...], PI,
        dimension_numbers=(((1,), (1,)), ((), ())),
        preferred_element_type=jnp.float32)


def _ln(x, g, b, eps=1e-5):
    mu = jnp.mean(x, axis=-1, keepdims=True)
    xc = x - mu
    var = jnp.mean(xc * xc, axis=-1, keepdims=True)
    return xc * jax.lax.rsqrt(var + eps) * g + b


def _out_body(res_ref, al_ref, v_ref, w1_ref, b1_ref, w2_ref, b2_ref,
              g_ref, bt_ref, out_ref):
    res = res_ref[...]
    rb = res.astype(jnp.bfloat16)
    PV = jnp.dot(rb, v_ref[...], preferred_element_type=jnp.float32)
    al = al_ref[...].T
    mo = (al[:, 0:1] * PV[:, 0:128] + al[:, 1:2] * PV[:, 128:256]
          + al[:, 2:3] * PV[:, 256:384] + al[:, 3:4] * PV[:, 384:512])
    g = g_ref[...]
    bt = bt_ref[...]
    x = _ln(mo + res, g, bt)
    h1 = jnp.dot(x.astype(jnp.bfloat16), w1_ref[...],
                 preferred_element_type=jnp.float32) + b1_ref[...]
    h1 = jnp.maximum(h1, 0.0)
    h2 = jnp.dot(h1.astype(jnp.bfloat16), w2_ref[...],
                 preferred_element_type=jnp.float32) + b2_ref[...]
    out_ref[...] = _ln(h2 + x, g, bt)


def _attention_logits(residue_h, inter_h, Mstack_bf, ET_bf):
    grid = (N // BA,)
    return pl.pallas_call(
        _att_body,
        grid=grid,
        in_specs=[
            pl.BlockSpec((BA, IN_DIM), lambda i: (i, 0)),
            pl.BlockSpec((BA, IN_DIM), lambda i: (i, 0)),
            pl.BlockSpec((IN_DIM, 4 * HID), lambda i: (0, 0)),
            pl.BlockSpec((HEAD, 4 * HID), lambda i: (0, 0)),
        ],
        out_specs=pl.BlockSpec((HEAD, BA), lambda i: (0, i)),
        out_shape=jax.ShapeDtypeStruct((HEAD, N), jnp.float32),
        compiler_params=pltpu.CompilerParams(
            dimension_semantics=("parallel",)),
    )(residue_h, inter_h, Mstack_bf, ET_bf)


def _output_block(residue_h, alpha, Vstack_bf, W1t_bf, b1r, W2t_bf, b2r,
                  gr, br):
    grid = (N // BC,)
    return pl.pallas_call(
        _out_body,
        grid=grid,
        in_specs=[
            pl.BlockSpec((BC, IN_DIM), lambda i: (i, 0)),
            pl.BlockSpec((HEAD, BC), lambda i: (0, i)),
            pl.BlockSpec((IN_DIM, 4 * HID), lambda i: (0, 0)),
            pl.BlockSpec((HID, 2 * HID), lambda i: (0, 0)),
            pl.BlockSpec((1, 2 * HID), lambda i: (0, 0)),
            pl.BlockSpec((2 * HID, HID), lambda i: (0, 0)),
            pl.BlockSpec((1, HID), lambda i: (0, 0)),
            pl.BlockSpec((1, IN_DIM), lambda i: (0, 0)),
            pl.BlockSpec((1, IN_DIM), lambda i: (0, 0)),
        ],
        out_specs=pl.BlockSpec((BC, IN_DIM), lambda i: (i, 0)),
        out_shape=jax.ShapeDtypeStruct((N, IN_DIM), jnp.float32),
        compiler_params=pltpu.CompilerParams(
            dimension_semantics=("parallel",)),
    )(residue_h, alpha, Vstack_bf, W1t_bf, b1r, W2t_bf, b2r, gr, br)


NC = 2
NS = 16
LANES = 16
GRP = 128
NG = N // GRP
SEGP = 40960
RED = SEGP // NS
CHG = 16

P1_BASE = NG // NS
P1_EXTRA = NG - P1_BASE * NS
P1_FULL = P1_BASE // CHG
P1_T0 = P1_BASE - P1_FULL * CHG

NW = NC * NS
P2_BASE = NG // NW
P2_EXTRA = NG - P2_BASE * NW
P2_FULL = P2_BASE // CHG
P2_T0 = P2_BASE - P2_FULL * CHG


def _sc_softmax_body(att_hbm, batch_hbm, alpha_hbm, denom, attc, bc, outc,
                     tmp, acc, shared):
    cid = lax.axis_index("c")
    sid = lax.axis_index("s")
    iot = lax.iota(jnp.int32, LANES)
    zero16 = jnp.zeros((LANES,), jnp.float32)

    def zero_body(i, _):
        denom[pl.ds(i * LANES, LANES)] = zero16
        return 0

    lax.fori_loop(0, SEGP // LANES, zero_body, 0)

    def p1_chunk(tok0, ntok):
        pltpu.sync_copy(att_hbm.at[:, pl.ds(tok0, ntok)],
                        attc.at[:, pl.ds(0, ntok)])
        pltpu.sync_copy(batch_hbm.at[pl.ds(tok0, ntok)], bc.at[pl.ds(0, ntok)])

        def vr(j, _):
            base = j * LANES
            b = bc[pl.ds(base, LANES)]
            bn = bc[pl.ds(base + 1, LANES)]
            is_end = (b != bn) | (iot == LANES - 1)
            is_mid_end = is_end & (iot != LANES - 1)
            b4 = b * HEAD
            bn4 = bn * HEAD
            for h in range(HEAD):
                e = jnp.exp(attc[h, pl.ds(base, LANES)])
                c = plsc.cumsum(e)
                plsc.addupdate_scatter(denom, [b4 + h], c, mask=is_end)
                plsc.addupdate_scatter(denom, [bn4 + h], -c, mask=is_mid_end)
            return 0

        lax.fori_loop(0, ntok // LANES, vr, 0)

    g0 = P1_BASE * sid + jnp.minimum(sid, P1_EXTRA)

    def chunk1(k, _):
        p1_chunk((g0 + k * CHG) * GRP, CHG * GRP)
        return 0

    lax.fori_loop(0, P1_FULL, chunk1, 0)
    tail0 = (g0 + P1_FULL * CHG) * GRP

    @pl.when(sid < P1_EXTRA)
    def _():
        p1_chunk(tail0, (P1_T0 + 1) * GRP)

    @pl.when(sid >= P1_EXTRA)
    def _():
        p1_chunk(tail0, P1_T0 * GRP)

    pltpu.sync_copy(denom, shared.at[sid])
    plsc.subcore_barrier()
    s0 = sid * RED
    pltpu.sync_copy(shared.at[0, pl.ds(s0, RED)], acc)

    def red(u, _):
        pltpu.sync_copy(shared.at[u, pl.ds(s0, RED)], tmp)

        def addv(i, _):
            sl = pl.ds(i * LANES, LANES)
            acc[sl] += tmp[sl]
            return 0

        lax.fori_loop(0, RED // LANES, addv, 0)
        return 0

    lax.fori_loop(1, NS, red, 0)
    pltpu.sync_copy(acc, shared.at[0, pl.ds(s0, RED)])
    plsc.subcore_barrier()
    pltpu.sync_copy(shared.at[0], denom)

    wid = cid * NS + sid

    def p2_chunk(tok0, ntok):
        pltpu.sync_copy(att_hbm.at[:, pl.ds(tok0, ntok)],
                        attc.at[:, pl.ds(0, ntok)])
        pltpu.sync_copy(batch_hbm.at[pl.ds(tok0, ntok)], bc.at[pl.ds(0, ntok)])

        def vr(j, _):
            base = j * LANES
            b = bc[pl.ds(base, LANES)]
            b4 = b * HEAD
            for h in range(HEAD):
                e = jnp.exp(attc[h, pl.ds(base, LANES)])
                d = plsc.load_gather(denom, [b4 + h])
                outc[h, pl.ds(base, LANES)] = e / d
            return 0

        lax.fori_loop(0, ntok // LANES, vr, 0)
        pltpu.sync_copy(outc.at[:, pl.ds(0, ntok)],
                        alpha_hbm.at[:, pl.ds(tok0, ntok)])

    w0 = P2_BASE * wid + jnp.minimum(wid, P2_EXTRA)

    def chunk2(k, _):
        p2_chunk((w0 + k * CHG) * GRP, CHG * GRP)
        return 0

    lax.fori_loop(0, P2_FULL, chunk2, 0)
    tail2 = (w0 + P2_FULL * CHG) * GRP

    @pl.when(wid < P2_EXTRA)
    def _():
        p2_chunk(tail2, (P2_T0 + 1) * GRP)

    @pl.when(wid >= P2_EXTRA)
    def _():
        p2_chunk(tail2, P2_T0 * GRP)


def _sc_softmax(attT, batch):
    mesh = plsc.VectorSubcoreMesh(core_axis_name="c", subcore_axis_name="s")
    return pl.kernel(
        _sc_softmax_body,
        out_type=jax.ShapeDtypeStruct((HEAD, N), jnp.float32),
        mesh=mesh,
        scratch_types=[
            pltpu.VMEM((SEGP,), jnp.float32),
            pltpu.VMEM((HEAD, (CHG + 1) * GRP), jnp.float32),
            pltpu.VMEM(((CHG + 1) * GRP + LANES,), jnp.int32),
            pltpu.VMEM((HEAD, (CHG + 1) * GRP), jnp.float32),
            pltpu.VMEM((RED,), jnp.float32),
            pltpu.VMEM((RED,), jnp.float32),
            pltpu.VMEM_SHARED((NS, SEGP), jnp.float32),
        ],
        compiler_params=pltpu.CompilerParams(needs_layout_passes=False),
    )(attT, batch)


def kernel(residue_h, inter_h, Wq, Wk, Wv, Wc, W1, b1, W2, b2, gamma, beta,
           batch):
    scale = jnp.sqrt(jnp.float32(1280.0))
    Mstack = jnp.concatenate(
        [Wq[i].T @ Wk[i] for i in range(HEAD)], axis=1) / scale
    Vstack = jnp.concatenate(
        [Wv[i].T @ Wc[:, i * HID:(i + 1) * HID].T for i in range(HEAD)],
        axis=1)
    ET = jnp.repeat(jnp.eye(HEAD, dtype=jnp.float32), HID, axis=1)

    attT = _attention_logits(residue_h, inter_h,
                             Mstack.astype(jnp.bfloat16),
                             ET.astype(jnp.bfloat16))

    alpha = _sc_softmax(attT, batch)

    return _output_block(residue_h, alpha,
                         Vstack.astype(jnp.bfloat16),
                         W1.T.astype(jnp.bfloat16),
                         b1.reshape(1, -1),
                         W2.T.astype(jnp.bfloat16),
                         b2.reshape(1, -1),
                         gamma.reshape(1, -1),
                         beta.reshape(1, -1))


# speedup vs baseline: 1.5623x; 1.0854x over previous
import functools

import jax
import jax.numpy as jnp
from jax import lax
from jax.experimental import pallas as pl
from jax.experimental.pallas import tpu as pltpu
from jax.experimental.pallas import tpu_sc as plsc

N = 320000
IN_DIM = 128
HID = 128
HEAD = 4
NUM_SEG = 10000

BA = 3200
BC = 3200


def _att_body(res_ref, int_ref, m_ref, et_ref, out_ref):
    rb = res_ref[...].astype(jnp.bfloat16)
    P = jnp.dot(rb, m_ref[...], preferred_element_type=jnp.float32)
    ib = int_ref[...].astype(jnp.bfloat16)
    i4 = jnp.concatenate([ib, ib, ib, ib], axis=1)
    PI = (P.astype(jnp.bfloat16) * i4)
    out_ref[...] = jax.lax.dot_general(
        et_ref[...], PI,
        dimension_numbers=(((1,), (1,)), ((), ())),
        preferred_element_type=jnp.float32)


def _ln(x, g, b, o, eps=1e-5):
    xb = x.astype(jnp.bfloat16)
    mu = jnp.dot(xb, o, preferred_element_type=jnp.float32)
    xc = x - mu
    xcb = xc.astype(jnp.bfloat16)
    var = jnp.dot(xcb * xcb, o, preferred_element_type=jnp.float32)
    return xc * jax.lax.rsqrt(var + eps) * g + b


def _out_body(res_ref, al_ref, v_ref, et_ref, o_ref, w1_ref, b1_ref, w2_ref,
              b2_ref, g_ref, bt_ref, out_ref):
    res = res_ref[...]
    rb = res.astype(jnp.bfloat16)
    PV = jnp.dot(rb, v_ref[...], preferred_element_type=jnp.float32)
    alT = jax.lax.dot_general(
        al_ref[...].astype(jnp.bfloat16), et_ref[...],
        dimension_numbers=(((0,), (0,)), ((), ())),
        preferred_element_type=jnp.float32)
    Z = alT * PV
    mo = (Z[:, 0:128] + Z[:, 128:256] + Z[:, 256:384] + Z[:, 384:512])
    g = g_ref[...]
    bt = bt_ref[...]
    o = o_ref[...]
    x = _ln(mo + res, g, bt, o)
    h1 = jnp.dot(x.astype(jnp.bfloat16), w1_ref[...],
                 preferred_element_type=jnp.float32) + b1_ref[...]
    h1 = jnp.maximum(h1, 0.0)
    h2 = jnp.dot(h1.astype(jnp.bfloat16), w2_ref[...],
                 preferred_element_type=jnp.float32) + b2_ref[...]
    out_ref[...] = _ln(h2 + x, g, bt, o)


def _attention_logits(residue_h, inter_h, Mstack_bf, ET_bf):
    grid = (N // BA,)
    return pl.pallas_call(
        _att_body,
        grid=grid,
        in_specs=[
            pl.BlockSpec((BA, IN_DIM), lambda i: (i, 0)),
            pl.BlockSpec((BA, IN_DIM), lambda i: (i, 0)),
            pl.BlockSpec((IN_DIM, 4 * HID), lambda i: (0, 0)),
            pl.BlockSpec((HEAD, 4 * HID), lambda i: (0, 0)),
        ],
        out_specs=pl.BlockSpec((HEAD, BA), lambda i: (0, i)),
        out_shape=jax.ShapeDtypeStruct((HEAD, N), jnp.float32),
        compiler_params=pltpu.CompilerParams(
            dimension_semantics=("parallel",)),
    )(residue_h, inter_h, Mstack_bf, ET_bf)


def _output_block(residue_h, alpha, Vstack_bf, ET_bf, Ones_bf, W1t_bf, b1r,
                  W2t_bf, b2r, gr, br):
    grid = (N // BC,)
    return pl.pallas_call(
        _out_body,
        grid=grid,
        in_specs=[
            pl.BlockSpec((BC, IN_DIM), lambda i: (i, 0)),
            pl.BlockSpec((HEAD, BC), lambda i: (0, i)),
            pl.BlockSpec((IN_DIM, 4 * HID), lambda i: (0, 0)),
            pl.BlockSpec((HEAD, 4 * HID), lambda i: (0, 0)),
            pl.BlockSpec((IN_DIM, IN_DIM), lambda i: (0, 0)),
            pl.BlockSpec((HID, 2 * HID), lambda i: (0, 0)),
            pl.BlockSpec((1, 2 * HID), lambda i: (0, 0)),
            pl.BlockSpec((2 * HID, HID), lambda i: (0, 0)),
            pl.BlockSpec((1, HID), lambda i: (0, 0)),
            pl.BlockSpec((1, IN_DIM), lambda i: (0, 0)),
            pl.BlockSpec((1, IN_DIM), lambda i: (0, 0)),
        ],
        out_specs=pl.BlockSpec((BC, IN_DIM), lambda i: (i, 0)),
        out_shape=jax.ShapeDtypeStruct((N, IN_DIM), jnp.float32),
        compiler_params=pltpu.CompilerParams(
            dimension_semantics=("parallel",)),
    )(residue_h, alpha, Vstack_bf, ET_bf, Ones_bf, W1t_bf, b1r, W2t_bf, b2r,
      gr, br)


NC = 2
NS = 16
LANES = 16
GRP = 128
NG = N // GRP
SEGP = 40960
RED = SEGP // NS
CHG = 16

P1_BASE = NG // NS
P1_EXTRA = NG - P1_BASE * NS
P1_FULL = P1_BASE // CHG
P1_T0 = P1_BASE - P1_FULL * CHG

NW = NC * NS
P2_BASE = NG // NW
P2_EXTRA = NG - P2_BASE * NW
P2_FULL = P2_BASE // CHG
P2_T0 = P2_BASE - P2_FULL * CHG


def _sc_softmax_body(att_hbm, batch_hbm, alpha_hbm, denom, attc, bc, outc,
                     tmp, acc, shared):
    cid = lax.axis_index("c")
    sid = lax.axis_index("s")
    iot = lax.iota(jnp.int32, LANES)
    zero16 = jnp.zeros((LANES,), jnp.float32)

    def zero_body(i, _):
        denom[pl.ds(i * LANES, LANES)] = zero16
        return 0

    lax.fori_loop(0, SEGP // LANES, zero_body, 0)

    def p1_chunk(tok0, ntok):
        pltpu.sync_copy(att_hbm.at[:, pl.ds(tok0, ntok)],
                        attc.at[:, pl.ds(0, ntok)])
        pltpu.sync_copy(batch_hbm.at[pl.ds(tok0, ntok)], bc.at[pl.ds(0, ntok)])

        def vr(j, _):
            base = j * LANES
            b = bc[pl.ds(base, LANES)]
            bn = bc[pl.ds(base + 1, LANES)]
            is_end = (b != bn) | (iot == LANES - 1)
            is_mid_end = is_end & (iot != LANES - 1)
            b4 = b * HEAD
            bn4 = bn * HEAD
            for h in range(HEAD):
                e = jnp.exp(attc[h, pl.ds(base, LANES)])
                c = plsc.cumsum(e)
                plsc.addupdate_scatter(denom, [b4 + h], c, mask=is_end)
                plsc.addupdate_scatter(denom, [bn4 + h], -c, mask=is_mid_end)
            return 0

        lax.fori_loop(0, ntok // LANES, vr, 0)

    g0 = P1_BASE * sid + jnp.minimum(sid, P1_EXTRA)

    def chunk1(k, _):
        p1_chunk((g0 + k * CHG) * GRP, CHG * GRP)
        return 0

    lax.fori_loop(0, P1_FULL, chunk1, 0)
    tail0 = (g0 + P1_FULL * CHG) * GRP

    @pl.when(sid < P1_EXTRA)
    def _():
        p1_chunk(tail0, (P1_T0 + 1) * GRP)

    @pl.when(sid >= P1_EXTRA)
    def _():
        p1_chunk(tail0, P1_T0 * GRP)

    pltpu.sync_copy(denom, shared.at[sid])
    plsc.subcore_barrier()
    s0 = sid * RED
    pltpu.sync_copy(shared.at[0, pl.ds(s0, RED)], acc)

    def red(u, _):
        pltpu.sync_copy(shared.at[u, pl.ds(s0, RED)], tmp)

        def addv(i, _):
            sl = pl.ds(i * LANES, LANES)
            acc[sl] += tmp[sl]
            return 0

        lax.fori_loop(0, RED // LANES, addv, 0)
        return 0

    lax.fori_loop(1, NS, red, 0)
    pltpu.sync_copy(acc, shared.at[0, pl.ds(s0, RED)])
    plsc.subcore_barrier()
    pltpu.sync_copy(shared.at[0], denom)

    wid = cid * NS + sid

    def p2_chunk(tok0, ntok):
        pltpu.sync_copy(att_hbm.at[:, pl.ds(tok0, ntok)],
                        attc.at[:, pl.ds(0, ntok)])
        pltpu.sync_copy(batch_hbm.at[pl.ds(tok0, ntok)], bc.at[pl.ds(0, ntok)])

        def vr(j, _):
            base = j * LANES
            b = bc[pl.ds(base, LANES)]
            b4 = b * HEAD
            for h in range(HEAD):
                e = jnp.exp(attc[h, pl.ds(base, LANES)])
                d = plsc.load_gather(denom, [b4 + h])
                outc[h, pl.ds(base, LANES)] = e / d
            return 0

        lax.fori_loop(0, ntok // LANES, vr, 0)
        pltpu.sync_copy(outc.at[:, pl.ds(0, ntok)],
                        alpha_hbm.at[:, pl.ds(tok0, ntok)])

    w0 = P2_BASE * wid + jnp.minimum(wid, P2_EXTRA)

    def chunk2(k, _):
        p2_chunk((w0 + k * CHG) * GRP, CHG * GRP)
        return 0

    lax.fori_loop(0, P2_FULL, chunk2, 0)
    tail2 = (w0 + P2_FULL * CHG) * GRP

    @pl.when(wid < P2_EXTRA)
    def _():
        p2_chunk(tail2, (P2_T0 + 1) * GRP)

    @pl.when(wid >= P2_EXTRA)
    def _():
        p2_chunk(tail2, P2_T0 * GRP)


def _sc_softmax(attT, batch):
    mesh = plsc.VectorSubcoreMesh(core_axis_name="c", subcore_axis_name="s")
    return pl.kernel(
        _sc_softmax_body,
        out_type=jax.ShapeDtypeStruct((HEAD, N), jnp.float32),
        mesh=mesh,
        scratch_types=[
            pltpu.VMEM((SEGP,), jnp.float32),
            pltpu.VMEM((HEAD, (CHG + 1) * GRP), jnp.float32),
            pltpu.VMEM(((CHG + 1) * GRP + LANES,), jnp.int32),
            pltpu.VMEM((HEAD, (CHG + 1) * GRP), jnp.float32),
            pltpu.VMEM((RED,), jnp.float32),
            pltpu.VMEM((RED,), jnp.float32),
            pltpu.VMEM_SHARED((NS, SEGP), jnp.float32),
        ],
        compiler_params=pltpu.CompilerParams(needs_layout_passes=False),
    )(attT, batch)


def kernel(residue_h, inter_h, Wq, Wk, Wv, Wc, W1, b1, W2, b2, gamma, beta,
           batch):
    scale = jnp.sqrt(jnp.float32(1280.0))
    Mstack = jnp.concatenate(
        [Wq[i].T @ Wk[i] for i in range(HEAD)], axis=1) / scale
    Vstack = jnp.concatenate(
        [Wv[i].T @ Wc[:, i * HID:(i + 1) * HID].T for i in range(HEAD)],
        axis=1)
    ET = jnp.repeat(jnp.eye(HEAD, dtype=jnp.float32), HID, axis=1)

    attT = _attention_logits(residue_h, inter_h,
                             Mstack.astype(jnp.bfloat16),
                             ET.astype(jnp.bfloat16))

    alpha = _sc_softmax(attT, batch)

    Ones = jnp.full((IN_DIM, IN_DIM), 1.0 / IN_DIM, dtype=jnp.float32)
    return _output_block(residue_h, alpha,
                         Vstack.astype(jnp.bfloat16),
                         ET.astype(jnp.bfloat16),
                         Ones.astype(jnp.bfloat16),
                         W1.T.astype(jnp.bfloat16),
                         b1.reshape(1, -1),
                         W2.T.astype(jnp.bfloat16),
                         b2.reshape(1, -1),
                         gamma.reshape(1, -1),
                         beta.reshape(1, -1))


# speedup vs baseline: 1.6361x; 1.0473x over previous
import functools

import jax
import jax.numpy as jnp
from jax import lax
from jax.experimental import pallas as pl
from jax.experimental.pallas import tpu as pltpu
from jax.experimental.pallas import tpu_sc as plsc

N = 320000
IN_DIM = 128
HID = 128
HEAD = 4
NUM_SEG = 10000

BA = 6400
BC = 6400


def _att_body(res_ref, int_ref, m_ref, et_ref, out_ref):
    rb = res_ref[...].astype(jnp.bfloat16)
    P = jnp.dot(rb, m_ref[...], preferred_element_type=jnp.float32)
    ib = int_ref[...].astype(jnp.bfloat16)
    i4 = jnp.concatenate([ib, ib, ib, ib], axis=1)
    PI = (P.astype(jnp.bfloat16) * i4)
    out_ref[...] = jax.lax.dot_general(
        et_ref[...], PI,
        dimension_numbers=(((1,), (1,)), ((), ())),
        preferred_element_type=jnp.float32)


def _ln(x, g, b, o2, eps=1e-5):
    xb = x.astype(jnp.bfloat16)
    X2 = jnp.concatenate([xb, xb * xb], axis=1)
    S = jnp.dot(X2, o2, preferred_element_type=jnp.float32)
    mu = S[:, 0:IN_DIM]
    var = S[:, IN_DIM:2 * IN_DIM] - mu * mu
    return (x - mu) * jax.lax.rsqrt(var + eps) * g + b


def _out_body(res_ref, al_ref, v_ref, et_ref, o_ref, w1_ref, b1_ref, w2_ref,
              b2_ref, g_ref, bt_ref, out_ref):
    res = res_ref[...]
    rb = res.astype(jnp.bfloat16)
    PV = jnp.dot(rb, v_ref[...], preferred_element_type=jnp.float32)
    alT = jax.lax.dot_general(
        al_ref[...].astype(jnp.bfloat16), et_ref[...],
        dimension_numbers=(((0,), (0,)), ((), ())),
        preferred_element_type=jnp.float32)
    Z = alT * PV
    mo = (Z[:, 0:128] + Z[:, 128:256] + Z[:, 256:384] + Z[:, 384:512])
    g = g_ref[...]
    bt = bt_ref[...]
    o = o_ref[...]
    x = _ln(mo + res, g, bt, o)
    h1 = jnp.dot(x.astype(jnp.bfloat16), w1_ref[...],
                 preferred_element_type=jnp.float32) + b1_ref[...]
    h1 = jnp.maximum(h1, 0.0)
    h2 = jnp.dot(h1.astype(jnp.bfloat16), w2_ref[...],
                 preferred_element_type=jnp.float32) + b2_ref[...]
    out_ref[...] = _ln(h2 + x, g, bt, o)


def _attention_logits(residue_h, inter_h, Mstack_bf, ET_bf):
    grid = (N // BA,)
    return pl.pallas_call(
        _att_body,
        grid=grid,
        in_specs=[
            pl.BlockSpec((BA, IN_DIM), lambda i: (i, 0)),
            pl.BlockSpec((BA, IN_DIM), lambda i: (i, 0)),
            pl.BlockSpec((IN_DIM, 4 * HID), lambda i: (0, 0)),
            pl.BlockSpec((HEAD, 4 * HID), lambda i: (0, 0)),
        ],
        out_specs=pl.BlockSpec((HEAD, BA), lambda i: (0, i)),
        out_shape=jax.ShapeDtypeStruct((HEAD, N), jnp.float32),
        compiler_params=pltpu.CompilerParams(
            dimension_semantics=("parallel",)),
    )(residue_h, inter_h, Mstack_bf, ET_bf)


def _output_block(residue_h, alpha, Vstack_bf, ET_bf, Ones_bf, W1t_bf, b1r,
                  W2t_bf, b2r, gr, br):
    grid = (N // BC,)
    return pl.pallas_call(
        _out_body,
        grid=grid,
        in_specs=[
            pl.BlockSpec((BC, IN_DIM), lambda i: (i, 0)),
            pl.BlockSpec((HEAD, BC), lambda i: (0, i)),
            pl.BlockSpec((IN_DIM, 4 * HID), lambda i: (0, 0)),
            pl.BlockSpec((HEAD, 4 * HID), lambda i: (0, 0)),
            pl.BlockSpec((2 * IN_DIM, 2 * IN_DIM), lambda i: (0, 0)),
            pl.BlockSpec((HID, 2 * HID), lambda i: (0, 0)),
            pl.BlockSpec((1, 2 * HID), lambda i: (0, 0)),
            pl.BlockSpec((2 * HID, HID), lambda i: (0, 0)),
            pl.BlockSpec((1, HID), lambda i: (0, 0)),
            pl.BlockSpec((1, IN_DIM), lambda i: (0, 0)),
            pl.BlockSpec((1, IN_DIM), lambda i: (0, 0)),
        ],
        out_specs=pl.BlockSpec((BC, IN_DIM), lambda i: (i, 0)),
        out_shape=jax.ShapeDtypeStruct((N, IN_DIM), jnp.float32),
        compiler_params=pltpu.CompilerParams(
            dimension_semantics=("parallel",)),
    )(residue_h, alpha, Vstack_bf, ET_bf, Ones_bf, W1t_bf, b1r, W2t_bf, b2r,
      gr, br)


NC = 2
NS = 16
LANES = 16
GRP = 128
NG = N // GRP
SEGP = 40960
RED = SEGP // NS
CHG = 16

P1_BASE = NG // NS
P1_EXTRA = NG - P1_BASE * NS
P1_FULL = P1_BASE // CHG
P1_T0 = P1_BASE - P1_FULL * CHG

NW = NC * NS
P2_BASE = NG // NW
P2_EXTRA = NG - P2_BASE * NW
P2_FULL = P2_BASE // CHG
P2_T0 = P2_BASE - P2_FULL * CHG


def _sc_softmax_body(att_hbm, batch_hbm, alpha_hbm, denom, attc, bc, outc,
                     tmp, acc, shared):
    cid = lax.axis_index("c")
    sid = lax.axis_index("s")
    iot = lax.iota(jnp.int32, LANES)
    zero16 = jnp.zeros((LANES,), jnp.float32)

    def zero_body(i, _):
        denom[pl.ds(i * LANES, LANES)] = zero16
        return 0

    lax.fori_loop(0, SEGP // LANES, zero_body, 0)

    def p1_chunk(tok0, ntok):
        pltpu.sync_copy(att_hbm.at[:, pl.ds(tok0, ntok)],
                        attc.at[:, pl.ds(0, ntok)])
        pltpu.sync_copy(batch_hbm.at[pl.ds(tok0, ntok)], bc.at[pl.ds(0, ntok)])

        def vr(j, _):
            base = j * LANES
            b = bc[pl.ds(base, LANES)]
            bn = bc[pl.ds(base + 1, LANES)]
            is_end = (b != bn) | (iot == LANES - 1)
            is_mid_end = is_end & (iot != LANES - 1)
            b4 = b * HEAD
            bn4 = bn * HEAD
            for h in range(HEAD):
                e = jnp.exp(attc[h, pl.ds(base, LANES)])
                c = plsc.cumsum(e)
                plsc.addupdate_scatter(denom, [b4 + h], c, mask=is_end)
                plsc.addupdate_scatter(denom, [bn4 + h], -c, mask=is_mid_end)
            return 0

        lax.fori_loop(0, ntok // LANES, vr, 0)

    g0 = P1_BASE * sid + jnp.minimum(sid, P1_EXTRA)

    def chunk1(k, _):
        p1_chunk((g0 + k * CHG) * GRP, CHG * GRP)
        return 0

    lax.fori_loop(0, P1_FULL, chunk1, 0)
    tail0 = (g0 + P1_FULL * CHG) * GRP

    @pl.when(sid < P1_EXTRA)
    def _():
        p1_chunk(tail0, (P1_T0 + 1) * GRP)

    @pl.when(sid >= P1_EXTRA)
    def _():
        p1_chunk(tail0, P1_T0 * GRP)

    pltpu.sync_copy(denom, shared.at[sid])
    plsc.subcore_barrier()
    s0 = sid * RED
    pltpu.sync_copy(shared.at[0, pl.ds(s0, RED)], acc)

    def red(u, _):
        pltpu.sync_copy(shared.at[u, pl.ds(s0, RED)], tmp)

        def addv(i, _):
            sl = pl.ds(i * LANES, LANES)
            acc[sl] += tmp[sl]
            return 0

        lax.fori_loop(0, RED // LANES, addv, 0)
        return 0

    lax.fori_loop(1, NS, red, 0)
    pltpu.sync_copy(acc, shared.at[0, pl.ds(s0, RED)])
    plsc.subcore_barrier()
    pltpu.sync_copy(shared.at[0], denom)

    wid = cid * NS + sid

    def p2_chunk(tok0, ntok):
        pltpu.sync_copy(att_hbm.at[:, pl.ds(tok0, ntok)],
                        attc.at[:, pl.ds(0, ntok)])
        pltpu.sync_copy(batch_hbm.at[pl.ds(tok0, ntok)], bc.at[pl.ds(0, ntok)])

        def vr(j, _):
            base = j * LANES
            b = bc[pl.ds(base, LANES)]
            b4 = b * HEAD
            for h in range(HEAD):
                e = jnp.exp(attc[h, pl.ds(base, LANES)])
                d = plsc.load_gather(denom, [b4 + h])
                outc[h, pl.ds(base, LANES)] = e / d
            return 0

        lax.fori_loop(0, ntok // LANES, vr, 0)
        pltpu.sync_copy(outc.at[:, pl.ds(0, ntok)],
                        alpha_hbm.at[:, pl.ds(tok0, ntok)])

    w0 = P2_BASE * wid + jnp.minimum(wid, P2_EXTRA)

    def chunk2(k, _):
        p2_chunk((w0 + k * CHG) * GRP, CHG * GRP)
        return 0

    lax.fori_loop(0, P2_FULL, chunk2, 0)
    tail2 = (w0 + P2_FULL * CHG) * GRP

    @pl.when(wid < P2_EXTRA)
    def _():
        p2_chunk(tail2, (P2_T0 + 1) * GRP)

    @pl.when(wid >= P2_EXTRA)
    def _():
        p2_chunk(tail2, P2_T0 * GRP)


def _sc_softmax(attT, batch):
    mesh = plsc.VectorSubcoreMesh(core_axis_name="c", subcore_axis_name="s")
    return pl.kernel(
        _sc_softmax_body,
        out_type=jax.ShapeDtypeStruct((HEAD, N), jnp.float32),
        mesh=mesh,
        scratch_types=[
            pltpu.VMEM((SEGP,), jnp.float32),
            pltpu.VMEM((HEAD, (CHG + 1) * GRP), jnp.float32),
            pltpu.VMEM(((CHG + 1) * GRP + LANES,), jnp.int32),
            pltpu.VMEM((HEAD, (CHG + 1) * GRP), jnp.float32),
            pltpu.VMEM((RED,), jnp.float32),
            pltpu.VMEM((RED,), jnp.float32),
            pltpu.VMEM_SHARED((NS, SEGP), jnp.float32),
        ],
        compiler_params=pltpu.CompilerParams(needs_layout_passes=False),
    )(attT, batch)


def kernel(residue_h, inter_h, Wq, Wk, Wv, Wc, W1, b1, W2, b2, gamma, beta,
           batch):
    scale = jnp.sqrt(jnp.float32(1280.0))
    Mstack = jnp.concatenate(
        [Wq[i].T @ Wk[i] for i in range(HEAD)], axis=1) / scale
    Vstack = jnp.concatenate(
        [Wv[i].T @ Wc[:, i * HID:(i + 1) * HID].T for i in range(HEAD)],
        axis=1)
    ET = jnp.repeat(jnp.eye(HEAD, dtype=jnp.float32), HID, axis=1)

    attT = _attention_logits(residue_h, inter_h,
                             Mstack.astype(jnp.bfloat16),
                             ET.astype(jnp.bfloat16))

    alpha = _sc_softmax(attT, batch)

    O1 = jnp.full((IN_DIM, IN_DIM), 1.0 / IN_DIM, dtype=jnp.float32)
    Zb = jnp.zeros((IN_DIM, IN_DIM), dtype=jnp.float32)
    Ones = jnp.block([[O1, Zb], [Zb, O1]])
    return _output_block(residue_h, alpha,
                         Vstack.astype(jnp.bfloat16),
                         ET.astype(jnp.bfloat16),
                         Ones.astype(jnp.bfloat16),
                         W1.T.astype(jnp.bfloat16),
                         b1.reshape(1, -1),
                         W2.T.astype(jnp.bfloat16),
                         b2.reshape(1, -1),
                         gamma.reshape(1, -1),
                         beta.reshape(1, -1))


# speedup vs baseline: 1.8535x; 1.1328x over previous
import functools

import jax
import jax.numpy as jnp
from jax import lax
from jax.experimental import pallas as pl
from jax.experimental.pallas import tpu as pltpu
from jax.experimental.pallas import tpu_sc as plsc

N = 320000
IN_DIM = 128
HID = 128
HEAD = 4
NUM_SEG = 10000

BA = 6400
BC = 6400


def _att_body(res_ref, int_ref, m_ref, et_ref, out_ref):
    rb = res_ref[...].astype(jnp.bfloat16)
    P = jnp.dot(rb, m_ref[...], preferred_element_type=jnp.float32)
    ib = int_ref[...].astype(jnp.bfloat16)
    i4 = jnp.concatenate([ib, ib, ib, ib], axis=1)
    PI = (P.astype(jnp.bfloat16) * i4)
    out_ref[...] = jax.lax.dot_general(
        et_ref[...], PI,
        dimension_numbers=(((1,), (1,)), ((), ())),
        preferred_element_type=jnp.float32)


def _ln(x, g, b, o2, eps=1e-5):
    xb = x.astype(jnp.bfloat16)
    X2 = jnp.concatenate([xb, xb * xb], axis=1)
    S = jnp.dot(X2, o2, preferred_element_type=jnp.float32)
    mu = S[:, 0:IN_DIM]
    var = S[:, IN_DIM:2 * IN_DIM] - mu * mu
    return (x - mu) * jax.lax.rsqrt(var + eps) * g + b


def _out_body(res_ref, al_ref, v_ref, et_ref, o_ref, w1_ref, b1_ref, w2_ref,
              b2_ref, g_ref, bt_ref, out_ref):
    res = res_ref[...]
    rb = res.astype(jnp.bfloat16)
    PV = jnp.dot(rb, v_ref[...], preferred_element_type=jnp.float32)
    alT = jax.lax.dot_general(
        al_ref[...].astype(jnp.bfloat16), et_ref[...],
        dimension_numbers=(((0,), (0,)), ((), ())),
        preferred_element_type=jnp.float32)
    Z = alT * PV
    mo = (Z[:, 0:128] + Z[:, 128:256] + Z[:, 256:384] + Z[:, 384:512])
    g = g_ref[...]
    bt = bt_ref[...]
    o = o_ref[...]
    x = _ln(mo + res, g, bt, o)
    h1 = jnp.dot(x.astype(jnp.bfloat16), w1_ref[...],
                 preferred_element_type=jnp.float32) + b1_ref[...]
    h1 = jnp.maximum(h1, 0.0)
    h2 = jnp.dot(h1.astype(jnp.bfloat16), w2_ref[...],
                 preferred_element_type=jnp.float32) + b2_ref[...]
    out_ref[...] = _ln(h2 + x, g, bt, o)


def _attention_logits(residue_h, inter_h, Mstack_bf, ET_bf):
    grid = (N // BA,)
    return pl.pallas_call(
        _att_body,
        grid=grid,
        in_specs=[
            pl.BlockSpec((BA, IN_DIM), lambda i: (i, 0)),
            pl.BlockSpec((BA, IN_DIM), lambda i: (i, 0)),
            pl.BlockSpec((IN_DIM, 4 * HID), lambda i: (0, 0)),
            pl.BlockSpec((HEAD, 4 * HID), lambda i: (0, 0)),
        ],
        out_specs=pl.BlockSpec((HEAD, BA), lambda i: (0, i)),
        out_shape=jax.ShapeDtypeStruct((HEAD, N), jnp.float32),
        compiler_params=pltpu.CompilerParams(
            dimension_semantics=("parallel",)),
    )(residue_h, inter_h, Mstack_bf, ET_bf)


def _output_block(residue_h, alpha, Vstack_bf, ET_bf, Ones_bf, W1t_bf, b1r,
                  W2t_bf, b2r, gr, br):
    grid = (N // BC,)
    return pl.pallas_call(
        _out_body,
        grid=grid,
        in_specs=[
            pl.BlockSpec((BC, IN_DIM), lambda i: (i, 0)),
            pl.BlockSpec((HEAD, BC), lambda i: (0, i)),
            pl.BlockSpec((IN_DIM, 4 * HID), lambda i: (0, 0)),
            pl.BlockSpec((HEAD, 4 * HID), lambda i: (0, 0)),
            pl.BlockSpec((2 * IN_DIM, 2 * IN_DIM), lambda i: (0, 0)),
            pl.BlockSpec((HID, 2 * HID), lambda i: (0, 0)),
            pl.BlockSpec((1, 2 * HID), lambda i: (0, 0)),
            pl.BlockSpec((2 * HID, HID), lambda i: (0, 0)),
            pl.BlockSpec((1, HID), lambda i: (0, 0)),
            pl.BlockSpec((1, IN_DIM), lambda i: (0, 0)),
            pl.BlockSpec((1, IN_DIM), lambda i: (0, 0)),
        ],
        out_specs=pl.BlockSpec((BC, IN_DIM), lambda i: (i, 0)),
        out_shape=jax.ShapeDtypeStruct((N, IN_DIM), jnp.float32),
        compiler_params=pltpu.CompilerParams(
            dimension_semantics=("parallel",)),
    )(residue_h, alpha, Vstack_bf, ET_bf, Ones_bf, W1t_bf, b1r, W2t_bf, b2r,
      gr, br)


NC = 2
NS = 16
LANES = 16
GRP = 128
NG = N // GRP
SEGP = 40960
RED = SEGP // NS
CHG = 16

P1_BASE = NG // NS
P1_EXTRA = NG - P1_BASE * NS
P1_FULL = P1_BASE // CHG
P1_T0 = P1_BASE - P1_FULL * CHG

NW = NC * NS
P2_BASE = NG // NW
P2_EXTRA = NG - P2_BASE * NW
P2_FULL = P2_BASE // CHG
P2_T0 = P2_BASE - P2_FULL * CHG


def _sc_softmax_body(att_hbm, batch_hbm, alpha_hbm, denom, attc, bc, outc,
                     tmp, acc, shared):
    cid = lax.axis_index("c")
    sid = lax.axis_index("s")
    iot = lax.iota(jnp.int32, LANES)
    zero16 = jnp.zeros((LANES,), jnp.float32)

    def zero_body(i, _):
        denom[pl.ds(i * LANES, LANES)] = zero16
        return 0

    lax.fori_loop(0, SEGP // LANES, zero_body, 0)

    def p1_chunk(tok0, ntok):
        pltpu.sync_copy(att_hbm.at[:, pl.ds(tok0, ntok)],
                        attc.at[:, pl.ds(0, ntok)])
        pltpu.sync_copy(batch_hbm.at[pl.ds(tok0, ntok)], bc.at[pl.ds(0, ntok)])

        @plsc.parallel_loop(0, ntok // LANES, unroll=4)
        def vr(j):
            base = j * LANES
            b = bc[pl.ds(base, LANES)]
            bn = bc[pl.ds(base + 1, LANES)]
            is_end = (b != bn) | (iot == LANES - 1)
            is_mid_end = is_end & (iot != LANES - 1)
            b4 = b * HEAD
            bn4 = bn * HEAD
            for h in range(HEAD):
                e = jnp.exp(attc[h, pl.ds(base, LANES)])
                c = plsc.cumsum(e)
                plsc.addupdate_scatter(denom, [b4 + h], c, mask=is_end)
                plsc.addupdate_scatter(denom, [bn4 + h], -c, mask=is_mid_end)

    g0 = P1_BASE * sid + jnp.minimum(sid, P1_EXTRA)

    def chunk1(k, _):
        p1_chunk((g0 + k * CHG) * GRP, CHG * GRP)
        return 0

    lax.fori_loop(0, P1_FULL, chunk1, 0)
    tail0 = (g0 + P1_FULL * CHG) * GRP

    @pl.when(sid < P1_EXTRA)
    def _():
        p1_chunk(tail0, (P1_T0 + 1) * GRP)

    @pl.when(sid >= P1_EXTRA)
    def _():
        p1_chunk(tail0, P1_T0 * GRP)

    pltpu.sync_copy(denom, shared.at[sid])
    plsc.subcore_barrier()
    s0 = sid * RED
    pltpu.sync_copy(shared.at[0, pl.ds(s0, RED)], acc)

    def red(u, _):
        pltpu.sync_copy(shared.at[u, pl.ds(s0, RED)], tmp)

        def addv(i, _):
            sl = pl.ds(i * LANES, LANES)
            acc[sl] += tmp[sl]
            return 0

        lax.fori_loop(0, RED // LANES, addv, 0)
        return 0

    lax.fori_loop(1, NS, red, 0)
    pltpu.sync_copy(acc, shared.at[0, pl.ds(s0, RED)])
    plsc.subcore_barrier()
    pltpu.sync_copy(shared.at[0], denom)

    wid = cid * NS + sid

    def p2_chunk(tok0, ntok):
        pltpu.sync_copy(att_hbm.at[:, pl.ds(tok0, ntok)],
                        attc.at[:, pl.ds(0, ntok)])
        pltpu.sync_copy(batch_hbm.at[pl.ds(tok0, ntok)], bc.at[pl.ds(0, ntok)])

        @plsc.parallel_loop(0, ntok // LANES, unroll=4)
        def vr(j):
            base = j * LANES
            b = bc[pl.ds(base, LANES)]
            b4 = b * HEAD
            for h in range(HEAD):
                e = jnp.exp(attc[h, pl.ds(base, LANES)])
                d = plsc.load_gather(denom, [b4 + h])
                outc[h, pl.ds(base, LANES)] = e / d
        pltpu.sync_copy(outc.at[:, pl.ds(0, ntok)],
                        alpha_hbm.at[:, pl.ds(tok0, ntok)])

    w0 = P2_BASE * wid + jnp.minimum(wid, P2_EXTRA)

    def chunk2(k, _):
        p2_chunk((w0 + k * CHG) * GRP, CHG * GRP)
        return 0

    lax.fori_loop(0, P2_FULL, chunk2, 0)
    tail2 = (w0 + P2_FULL * CHG) * GRP

    @pl.when(wid < P2_EXTRA)
    def _():
        p2_chunk(tail2, (P2_T0 + 1) * GRP)

    @pl.when(wid >= P2_EXTRA)
    def _():
        p2_chunk(tail2, P2_T0 * GRP)


def _sc_softmax(attT, batch):
    mesh = plsc.VectorSubcoreMesh(core_axis_name="c", subcore_axis_name="s")
    return pl.kernel(
        _sc_softmax_body,
        out_type=jax.ShapeDtypeStruct((HEAD, N), jnp.float32),
        mesh=mesh,
        scratch_types=[
            pltpu.VMEM((SEGP,), jnp.float32),
            pltpu.VMEM((HEAD, (CHG + 1) * GRP), jnp.float32),
            pltpu.VMEM(((CHG + 1) * GRP + LANES,), jnp.int32),
            pltpu.VMEM((HEAD, (CHG + 1) * GRP), jnp.float32),
            pltpu.VMEM((RED,), jnp.float32),
            pltpu.VMEM((RED,), jnp.float32),
            pltpu.VMEM_SHARED((NS, SEGP), jnp.float32),
        ],
        compiler_params=pltpu.CompilerParams(needs_layout_passes=False),
    )(attT, batch)


def kernel(residue_h, inter_h, Wq, Wk, Wv, Wc, W1, b1, W2, b2, gamma, beta,
           batch):
    scale = jnp.sqrt(jnp.float32(1280.0))
    Mstack = jnp.concatenate(
        [Wq[i].T @ Wk[i] for i in range(HEAD)], axis=1) / scale
    Vstack = jnp.concatenate(
        [Wv[i].T @ Wc[:, i * HID:(i + 1) * HID].T for i in range(HEAD)],
        axis=1)
    ET = jnp.repeat(jnp.eye(HEAD, dtype=jnp.float32), HID, axis=1)

    attT = _attention_logits(residue_h, inter_h,
                             Mstack.astype(jnp.bfloat16),
                             ET.astype(jnp.bfloat16))

    alpha = _sc_softmax(attT, batch)

    O1 = jnp.full((IN_DIM, IN_DIM), 1.0 / IN_DIM, dtype=jnp.float32)
    Zb = jnp.zeros((IN_DIM, IN_DIM), dtype=jnp.float32)
    Ones = jnp.block([[O1, Zb], [Zb, O1]])
    return _output_block(residue_h, alpha,
                         Vstack.astype(jnp.bfloat16),
                         ET.astype(jnp.bfloat16),
                         Ones.astype(jnp.bfloat16),
                         W1.T.astype(jnp.bfloat16),
                         b1.reshape(1, -1),
                         W2.T.astype(jnp.bfloat16),
                         b2.reshape(1, -1),
                         gamma.reshape(1, -1),
                         beta.reshape(1, -1))


# speedup vs baseline: 1.9021x; 1.0263x over previous
import functools

import jax
import jax.numpy as jnp
from jax import lax
from jax.experimental import pallas as pl
from jax.experimental.pallas import tpu as pltpu
from jax.experimental.pallas import tpu_sc as plsc

N = 320000
IN_DIM = 128
HID = 128
HEAD = 4
NUM_SEG = 10000

BA = 6400
BC = 6400


def _att_body(res_ref, int_ref, m_ref, et_ref, out_ref):
    rb = res_ref[...].astype(jnp.bfloat16)
    P = jnp.dot(rb, m_ref[...], preferred_element_type=jnp.float32)
    ib = int_ref[...].astype(jnp.bfloat16)
    i4 = jnp.concatenate([ib, ib, ib, ib], axis=1)
    PI = (P.astype(jnp.bfloat16) * i4)
    out_ref[...] = jax.lax.dot_general(
        et_ref[...], PI,
        dimension_numbers=(((1,), (1,)), ((), ())),
        preferred_element_type=jnp.float32)


def _ln(x, g, b, o2, eps=1e-5):
    xb = x.astype(jnp.bfloat16)
    X2 = jnp.concatenate([xb, xb * xb], axis=1)
    S = jnp.dot(X2, o2, preferred_element_type=jnp.float32)
    mu = S[:, 0:IN_DIM]
    var = S[:, IN_DIM:2 * IN_DIM] - mu * mu
    return (x - mu) * jax.lax.rsqrt(var + eps) * g + b


def _out_body(res_ref, al_ref, v_ref, et_ref, o_ref, w1_ref, b1_ref, w2_ref,
              b2_ref, g_ref, bt_ref, out_ref):
    res = res_ref[...]
    rb = res.astype(jnp.bfloat16)
    PV = jnp.dot(rb, v_ref[...], preferred_element_type=jnp.float32)
    alT = jax.lax.dot_general(
        al_ref[...].astype(jnp.bfloat16), et_ref[...],
        dimension_numbers=(((0,), (0,)), ((), ())),
        preferred_element_type=jnp.float32)
    Z = alT * PV
    mo = (Z[:, 0:128] + Z[:, 128:256] + Z[:, 256:384] + Z[:, 384:512])
    g = g_ref[...]
    bt = bt_ref[...]
    o = o_ref[...]
    x = _ln(mo + res, g, bt, o)
    h1 = jnp.dot(x.astype(jnp.bfloat16), w1_ref[...],
                 preferred_element_type=jnp.float32) + b1_ref[...]
    h1 = jnp.maximum(h1, 0.0)
    h2 = jnp.dot(h1.astype(jnp.bfloat16), w2_ref[...],
                 preferred_element_type=jnp.float32) + b2_ref[...]
    out_ref[...] = _ln(h2 + x, g, bt, o)


def _attention_logits(residue_h, inter_h, Mstack_bf, ET_bf):
    grid = (N // BA,)
    return pl.pallas_call(
        _att_body,
        grid=grid,
        in_specs=[
            pl.BlockSpec((BA, IN_DIM), lambda i: (i, 0)),
            pl.BlockSpec((BA, IN_DIM), lambda i: (i, 0)),
            pl.BlockSpec((IN_DIM, 4 * HID), lambda i: (0, 0)),
            pl.BlockSpec((HEAD, 4 * HID), lambda i: (0, 0)),
        ],
        out_specs=pl.BlockSpec((HEAD, BA), lambda i: (0, i)),
        out_shape=jax.ShapeDtypeStruct((HEAD, N), jnp.float32),
        compiler_params=pltpu.CompilerParams(
            dimension_semantics=("parallel",)),
    )(residue_h, inter_h, Mstack_bf, ET_bf)


def _output_block(residue_h, alpha, Vstack_bf, ET_bf, Ones_bf, W1t_bf, b1r,
                  W2t_bf, b2r, gr, br):
    grid = (N // BC,)
    return pl.pallas_call(
        _out_body,
        grid=grid,
        in_specs=[
            pl.BlockSpec((BC, IN_DIM), lambda i: (i, 0)),
            pl.BlockSpec((HEAD, BC), lambda i: (0, i)),
            pl.BlockSpec((IN_DIM, 4 * HID), lambda i: (0, 0)),
            pl.BlockSpec((HEAD, 4 * HID), lambda i: (0, 0)),
            pl.BlockSpec((2 * IN_DIM, 2 * IN_DIM), lambda i: (0, 0)),
            pl.BlockSpec((HID, 2 * HID), lambda i: (0, 0)),
            pl.BlockSpec((1, 2 * HID), lambda i: (0, 0)),
            pl.BlockSpec((2 * HID, HID), lambda i: (0, 0)),
            pl.BlockSpec((1, HID), lambda i: (0, 0)),
            pl.BlockSpec((1, IN_DIM), lambda i: (0, 0)),
            pl.BlockSpec((1, IN_DIM), lambda i: (0, 0)),
        ],
        out_specs=pl.BlockSpec((BC, IN_DIM), lambda i: (i, 0)),
        out_shape=jax.ShapeDtypeStruct((N, IN_DIM), jnp.float32),
        compiler_params=pltpu.CompilerParams(
            dimension_semantics=("parallel",)),
    )(residue_h, alpha, Vstack_bf, ET_bf, Ones_bf, W1t_bf, b1r, W2t_bf, b2r,
      gr, br)


NC = 2
NS = 16
LANES = 16
GRP = 128
NG = N // GRP
SEGP = 40960
RED = SEGP // NS
CHG = 16

P1_BASE = NG // NS
P1_EXTRA = NG - P1_BASE * NS
P1_FULL = P1_BASE // CHG
P1_T0 = P1_BASE - P1_FULL * CHG

NW = NC * NS
P2_BASE = NG // NW
P2_EXTRA = NG - P2_BASE * NW
P2_FULL = P2_BASE // CHG
P2_T0 = P2_BASE - P2_FULL * CHG


def _sc_softmax_body(att_hbm, batch_hbm, alpha_hbm, denom, attc, bc, outc,
                     tmp, acc, shared):
    cid = lax.axis_index("c")
    sid = lax.axis_index("s")
    iot = lax.iota(jnp.int32, LANES)
    zero16 = jnp.zeros((LANES,), jnp.float32)

    def zero_body(i, _):
        denom[pl.ds(i * LANES, LANES)] = zero16
        return 0

    lax.fori_loop(0, SEGP // LANES, zero_body, 0)

    def p1_chunk(tok0, ntok):
        pltpu.sync_copy(att_hbm.at[:, pl.ds(tok0, ntok)],
                        attc.at[:, pl.ds(0, ntok)])
        pltpu.sync_copy(batch_hbm.at[pl.ds(tok0, ntok)], bc.at[pl.ds(0, ntok)])

        @plsc.parallel_loop(0, ntok // LANES, unroll=8)
        def vr(j):
            base = j * LANES
            b = bc[pl.ds(base, LANES)]
            bn = bc[pl.ds(base + 1, LANES)]
            is_end = (b != bn) | (iot == LANES - 1)
            is_mid_end = is_end & (iot != LANES - 1)
            b4 = b * HEAD
            bn4 = bn * HEAD
            for h in range(HEAD):
                e = jnp.exp(attc[h, pl.ds(base, LANES)])
                c = plsc.cumsum(e)
                plsc.addupdate_scatter(denom, [b4 + h], c, mask=is_end)
                plsc.addupdate_scatter(denom, [bn4 + h], -c, mask=is_mid_end)

    g0 = P1_BASE * sid + jnp.minimum(sid, P1_EXTRA)

    def chunk1(k, _):
        p1_chunk((g0 + k * CHG) * GRP, CHG * GRP)
        return 0

    lax.fori_loop(0, P1_FULL, chunk1, 0)
    tail0 = (g0 + P1_FULL * CHG) * GRP

    @pl.when(sid < P1_EXTRA)
    def _():
        p1_chunk(tail0, (P1_T0 + 1) * GRP)

    @pl.when(sid >= P1_EXTRA)
    def _():
        p1_chunk(tail0, P1_T0 * GRP)

    pltpu.sync_copy(denom, shared.at[sid])
    plsc.subcore_barrier()
    s0 = sid * RED
    pltpu.sync_copy(shared.at[0, pl.ds(s0, RED)], acc)

    def red(u, _):
        pltpu.sync_copy(shared.at[u, pl.ds(s0, RED)], tmp)

        def addv(i, _):
            sl = pl.ds(i * LANES, LANES)
            acc[sl] += tmp[sl]
            return 0

        lax.fori_loop(0, RED // LANES, addv, 0)
        return 0

    lax.fori_loop(1, NS, red, 0)
    pltpu.sync_copy(acc, shared.at[0, pl.ds(s0, RED)])
    plsc.subcore_barrier()
    pltpu.sync_copy(shared.at[0], denom)

    wid = cid * NS + sid

    def p2_chunk(tok0, ntok):
        pltpu.sync_copy(att_hbm.at[:, pl.ds(tok0, ntok)],
                        attc.at[:, pl.ds(0, ntok)])
        pltpu.sync_copy(batch_hbm.at[pl.ds(tok0, ntok)], bc.at[pl.ds(0, ntok)])

        @plsc.parallel_loop(0, ntok // LANES, unroll=8)
        def vr(j):
            base = j * LANES
            b = bc[pl.ds(base, LANES)]
            b4 = b * HEAD
            for h in range(HEAD):
                e = jnp.exp(attc[h, pl.ds(base, LANES)])
                d = plsc.load_gather(denom, [b4 + h])
                outc[h, pl.ds(base, LANES)] = e / d
        pltpu.sync_copy(outc.at[:, pl.ds(0, ntok)],
                        alpha_hbm.at[:, pl.ds(tok0, ntok)])

    w0 = P2_BASE * wid + jnp.minimum(wid, P2_EXTRA)

    def chunk2(k, _):
        p2_chunk((w0 + k * CHG) * GRP, CHG * GRP)
        return 0

    lax.fori_loop(0, P2_FULL, chunk2, 0)
    tail2 = (w0 + P2_FULL * CHG) * GRP

    @pl.when(wid < P2_EXTRA)
    def _():
        p2_chunk(tail2, (P2_T0 + 1) * GRP)

    @pl.when(wid >= P2_EXTRA)
    def _():
        p2_chunk(tail2, P2_T0 * GRP)


def _sc_softmax(attT, batch):
    mesh = plsc.VectorSubcoreMesh(core_axis_name="c", subcore_axis_name="s")
    return pl.kernel(
        _sc_softmax_body,
        out_type=jax.ShapeDtypeStruct((HEAD, N), jnp.float32),
        mesh=mesh,
        scratch_types=[
            pltpu.VMEM((SEGP,), jnp.float32),
            pltpu.VMEM((HEAD, (CHG + 1) * GRP), jnp.float32),
            pltpu.VMEM(((CHG + 1) * GRP + LANES,), jnp.int32),
            pltpu.VMEM((HEAD, (CHG + 1) * GRP), jnp.float32),
            pltpu.VMEM((RED,), jnp.float32),
            pltpu.VMEM((RED,), jnp.float32),
            pltpu.VMEM_SHARED((NS, SEGP), jnp.float32),
        ],
        compiler_params=pltpu.CompilerParams(needs_layout_passes=False),
    )(attT, batch)


def kernel(residue_h, inter_h, Wq, Wk, Wv, Wc, W1, b1, W2, b2, gamma, beta,
           batch):
    scale = jnp.sqrt(jnp.float32(1280.0))
    Mstack = jnp.concatenate(
        [Wq[i].T @ Wk[i] for i in range(HEAD)], axis=1) / scale
    Vstack = jnp.concatenate(
        [Wv[i].T @ Wc[:, i * HID:(i + 1) * HID].T for i in range(HEAD)],
        axis=1)
    ET = jnp.repeat(jnp.eye(HEAD, dtype=jnp.float32), HID, axis=1)

    attT = _attention_logits(residue_h, inter_h,
                             Mstack.astype(jnp.bfloat16),
                             ET.astype(jnp.bfloat16))

    alpha = _sc_softmax(attT, batch)

    O1 = jnp.full((IN_DIM, IN_DIM), 1.0 / IN_DIM, dtype=jnp.float32)
    Zb = jnp.zeros((IN_DIM, IN_DIM), dtype=jnp.float32)
    Ones = jnp.block([[O1, Zb], [Zb, O1]])
    return _output_block(residue_h, alpha,
                         Vstack.astype(jnp.bfloat16),
                         ET.astype(jnp.bfloat16),
                         Ones.astype(jnp.bfloat16),
                         W1.T.astype(jnp.bfloat16),
                         b1.reshape(1, -1),
                         W2.T.astype(jnp.bfloat16),
                         b2.reshape(1, -1),
                         gamma.reshape(1, -1),
                         beta.reshape(1, -1))
